# baseline (speedup 1.0000x reference)
import jax
import jax.numpy as jnp
from jax.experimental import pallas as pl

H = 128
_BLK = 1024


def _lrelu(x):
    return jnp.where(x >= 0, x, 0.2 * x)


def _body(audio_ref, text_ref, Wa_ref, ba_ref, Wt_ref, bt_ref,
          W1_ref, as1_ref, ad1_ref, b1_ref,
          W2_ref, as2_ref, ad2_ref, b2_ref,
          wf_ref, w2f_ref, Wfc_ref, bfc_ref, out_ref):
    f32 = jnp.float32
    bf16 = jnp.bfloat16

    def mm(a, b_ref):
        return jnp.dot(a.astype(bf16), b_ref[:], preferred_element_type=f32)

    xa = jnp.maximum(mm(audio_ref[:], Wa_ref) + ba_ref[:], 0.0)
    xt = jnp.maximum(mm(text_ref[:], Wt_ref) + bt_ref[:], 0.0)
    ga = mm(xa, W1_ref)
    gt = mm(xt, W1_ref)
    ya = jnp.maximum(ga + b1_ref[:], 0.0)
    yt = jnp.maximum(gt + b1_ref[:], 0.0)
    yab, ytb = ya.astype(bf16), yt.astype(bf16)
    zar = jnp.dot(yab, W2_ref[:], preferred_element_type=f32)
    ztr = jnp.dot(ytb, W2_ref[:], preferred_element_type=f32)
    la = jnp.dot(yab, w2f_ref[:], preferred_element_type=f32)
    lt = jnp.dot(ytb, w2f_ref[:], preferred_element_type=f32)
    wa = jax.nn.sigmoid(la[:, 0:1] - lt[:, 0:1])
    za = zar + b2_ref[:]
    zt = ztr + b2_ref[:]
    fused = wa * za + (1.0 - wa) * zt
    out_ref[:] = mm(fused, Wfc_ref) + bfc_ref[:]

    @pl.when(pl.program_id(0) == 0)
    def _fixup():
        ga0 = ga[0:1, :]
        gt0 = gt[0:1, :]
        mix_a, mix_t = [], []
        for h in range(2):
            sl = slice(h * H, (h + 1) * H)
            gah, gth = ga0[:, sl], gt0[:, sl]
            sv = as1_ref[h:h + 1, :]
            dv = ad1_ref[h:h + 1, :]
            asrc_a = jnp.sum(gah * sv, axis=1, keepdims=True)
            asrc_t = jnp.sum(gth * sv, axis=1, keepdims=True)
            adst_a = jnp.sum(gah * dv, axis=1, keepdims=True)
            adst_t = jnp.sum(gth * dv, axis=1, keepdims=True)
            al_s = _lrelu(asrc_a + adst_a)
            al_x = _lrelu(asrc_t + adst_a)
            m = jnp.maximum(al_s, al_x)
            es, ex = jnp.exp(al_s - m), jnp.exp(al_x - m)
            mix_a.append((es * gah + ex * gth) / (es + ex))
            bl_s = _lrelu(asrc_t + adst_t)
            bl_x = _lrelu(asrc_a + adst_t)
            m2 = jnp.maximum(bl_s, bl_x)
            fs, fx = jnp.exp(bl_s - m2), jnp.exp(bl_x - m2)
            mix_t.append((fs * gth + fx * gah) / (fs + fx))
        ya0 = jnp.maximum(jnp.concatenate(mix_a, axis=1) + b1_ref[:], 0.0)
        yt0 = jnp.maximum(jnp.concatenate(mix_t, axis=1) + b1_ref[:], 0.0)
        za0r = mm(ya0, W2_ref)
        zt0r = mm(yt0, W2_ref)
        s2, d2 = as2_ref[:], ad2_ref[:]
        asrc_a2 = jnp.sum(za0r * s2, axis=1, keepdims=True)
        asrc_t2 = jnp.sum(zt0r * s2, axis=1, keepdims=True)
        adst_a2 = jnp.sum(za0r * d2, axis=1, keepdims=True)
        adst_t2 = jnp.sum(zt0r * d2, axis=1, keepdims=True)
        al_s = _lrelu(asrc_a2 + adst_a2)
        al_x = _lrelu(asrc_t2 + adst_a2)
        m = jnp.maximum(al_s, al_x)
        es, ex = jnp.exp(al_s - m), jnp.exp(al_x - m)
        za0 = (es * za0r + ex * zt0r) / (es + ex) + b2_ref[:]
        bl_s = _lrelu(asrc_t2 + adst_t2)
        bl_x = _lrelu(asrc_a2 + adst_t2)
        m2 = jnp.maximum(bl_s, bl_x)
        fs, fx = jnp.exp(bl_s - m2), jnp.exp(bl_x - m2)
        zt0 = (fs * zt0r + fx * za0r) / (fs + fx) + b2_ref[:]
        la0 = jnp.sum(za0 * wf_ref[:], axis=1, keepdims=True)
        lt0 = jnp.sum(zt0 * wf_ref[:], axis=1, keepdims=True)
        wa0 = jax.nn.sigmoid(la0 - lt0)
        fused0 = wa0 * za0 + (1.0 - wa0) * zt0
        out_ref[0:1, :] = mm(fused0, Wfc_ref) + bfc_ref[:]


def kernel(audio_stats, text_stats, Wa, ba, Wt, bt, W1, att_src1, att_dst1, b1,
           W2, att_src2, att_dst2, b2, Wf, bf, Wfc, bfc):
    n = audio_stats.shape[0]
    bf16 = jnp.bfloat16
    row = lambda v: v.reshape(1, -1)
    rep = lambda shape: pl.BlockSpec(shape, lambda i: (0, 0))
    w2f = jnp.pad(W2 @ Wf, ((0, 0), (0, 7))).astype(bf16)
    return pl.pallas_call(
        _body,
        grid=(n // _BLK,),
        in_specs=[
            pl.BlockSpec((_BLK, H), lambda i: (i, 0)),
            pl.BlockSpec((_BLK, H), lambda i: (i, 0)),
            rep((H, H)), rep((1, H)), rep((H, H)), rep((1, H)),
            rep((H, 2 * H)), rep((2, H)), rep((2, H)), rep((1, 2 * H)),
            rep((2 * H, H)), rep((1, H)), rep((1, H)), rep((1, H)),
            rep((1, H)), rep((2 * H, 8)), rep((H, H)), rep((1, H)),
        ],
        out_specs=pl.BlockSpec((_BLK, H), lambda i: (i, 0)),
        out_shape=jax.ShapeDtypeStruct((n, H), jnp.float32),
    )(audio_stats, text_stats, Wa.astype(bf16), row(ba), Wt.astype(bf16), row(bt),
      W1.astype(bf16), att_src1, att_dst1, row(b1),
      W2.astype(bf16), att_src2, att_dst2, row(b2),
      Wf.T.reshape(1, H), w2f, Wfc.astype(bf16), row(bfc))


# speedup vs baseline: 1.8575x; 1.8575x over previous
import jax
import jax.numpy as jnp
from jax.experimental import pallas as pl

H = 128
_BLK = 2048


def _lrelu(x):
    return jnp.where(x >= 0, x, 0.2 * x)


def _mm(a, b_ref):
    return jnp.dot(a, b_ref[:], preferred_element_type=jnp.float32)


def _mix2(va, vt, sa_a, sa_t, da_a, da_t):
    al_s = _lrelu(sa_a + da_a)
    al_x = _lrelu(sa_t + da_a)
    m = jnp.maximum(al_s, al_x)
    es, ex = jnp.exp(al_s - m), jnp.exp(al_x - m)
    out_a = (es * va + ex * vt) / (es + ex)
    bl_s = _lrelu(sa_t + da_t)
    bl_x = _lrelu(sa_a + da_t)
    m2 = jnp.maximum(bl_s, bl_x)
    fs, fx = jnp.exp(bl_s - m2), jnp.exp(bl_x - m2)
    out_t = (fs * vt + fx * va) / (fs + fx)
    return out_a, out_t


def _body(audio_ref, text_ref, Wa_ref, ba_ref, Wt_ref, bt_ref,
          W1_ref, as1_ref, ad1_ref, b1_ref,
          W2_ref, as2_ref, ad2_ref, b2_ref,
          wf_ref, Wfc_ref, bfc_ref, out_ref):
    blk = audio_ref.shape[0]
    xa = jnp.maximum(_mm(audio_ref[:], Wa_ref) + ba_ref[:], 0.0)
    xt = jnp.maximum(_mm(text_ref[:], Wt_ref) + bt_ref[:], 0.0)
    ga = _mm(xa, W1_ref)
    gt = _mm(xt, W1_ref)

    row0 = (jax.lax.broadcasted_iota(jnp.int32, (blk, 1), 0) == 0) & (
        pl.program_id(0) == 0)

    ga0, gt0 = ga[0:1, :], gt[0:1, :]
    mix_a, mix_t = [], []
    for h in range(2):
        sl = slice(h * H, (h + 1) * H)
        gah, gth = ga0[:, sl], gt0[:, sl]
        sv = as1_ref[h:h + 1, :]
        dv = ad1_ref[h:h + 1, :]
        ma, mt = _mix2(
            gah, gth,
            jnp.sum(gah * sv, axis=1, keepdims=True),
            jnp.sum(gth * sv, axis=1, keepdims=True),
            jnp.sum(gah * dv, axis=1, keepdims=True),
            jnp.sum(gth * dv, axis=1, keepdims=True))
        mix_a.append(ma)
        mix_t.append(mt)
    ga = jnp.where(row0, jnp.concatenate(mix_a, axis=1), ga)
    gt = jnp.where(row0, jnp.concatenate(mix_t, axis=1), gt)

    ya = jnp.maximum(ga + b1_ref[:], 0.0)
    yt = jnp.maximum(gt + b1_ref[:], 0.0)
    zar = _mm(ya, W2_ref)
    ztr = _mm(yt, W2_ref)

    za0, zt0 = zar[0:1, :], ztr[0:1, :]
    s2, d2 = as2_ref[:], ad2_ref[:]
    ma, mt = _mix2(
        za0, zt0,
        jnp.sum(za0 * s2, axis=1, keepdims=True),
        jnp.sum(zt0 * s2, axis=1, keepdims=True),
        jnp.sum(za0 * d2, axis=1, keepdims=True),
        jnp.sum(zt0 * d2, axis=1, keepdims=True))
    zar = jnp.where(row0, ma, zar)
    ztr = jnp.where(row0, mt, ztr)

    diff = zar - ztr
    lml = jnp.sum(diff * wf_ref[:], axis=1, keepdims=True)
    wa = jax.nn.sigmoid(lml)
    fused = (ztr + b2_ref[:]) + wa * diff
    out_ref[:] = _mm(fused, Wfc_ref) + bfc_ref[:]


def kernel(audio_stats, text_stats, Wa, ba, Wt, bt, W1, att_src1, att_dst1, b1,
           W2, att_src2, att_dst2, b2, Wf, bf, Wfc, bfc):
    n = audio_stats.shape[0]
    row = lambda v: v.reshape(1, -1)
    rep = lambda shape: pl.BlockSpec(shape, lambda i: (0, 0))
    return pl.pallas_call(
        _body,
        grid=(n // _BLK,),
        in_specs=[
            pl.BlockSpec((_BLK, H), lambda i: (i, 0)),
            pl.BlockSpec((_BLK, H), lambda i: (i, 0)),
            rep((H, H)), rep((1, H)), rep((H, H)), rep((1, H)),
            rep((H, 2 * H)), rep((2, H)), rep((2, H)), rep((1, 2 * H)),
            rep((2 * H, H)), rep((1, H)), rep((1, H)), rep((1, H)),
            rep((1, H)), rep((H, H)), rep((1, H)),
        ],
        out_specs=pl.BlockSpec((_BLK, H), lambda i: (i, 0)),
        out_shape=jax.ShapeDtypeStruct((n, H), jnp.float32),
    )(audio_stats, text_stats, Wa, row(ba), Wt, row(bt),
      W1, att_src1, att_dst1, row(b1),
      W2, att_src2, att_dst2, row(b2),
      Wf.T.reshape(1, H), Wfc, row(bfc))
